# Initial kernel scaffold; baseline (speedup 1.0000x reference)
import jax
import jax.numpy as jnp
from jax.experimental import pallas as pl


def kernel(x, edge_index, mask, W1, b1, W2, b2, W3, b3, Dw1, Db1, Dw2, Db2, Dw3, Db3):
    raise NotImplementedError("write your pallas kernel here")


# speedup vs baseline: 2.4784x; 2.4784x over previous
import jax
import jax.numpy as jnp
from jax import lax
from jax.experimental import pallas as pl
from jax.experimental.pallas import tpu as pltpu

N = 10000
D = 128
H = 128
E = 320000

NC = 2
L = 16
N_ACC = 10112

RB = 1000
GRID = N // RB


def _tc_first_body(x_ref, w_ref, degp_ref, g_ref, dinv_ref):
    deg = degp_ref[0] + degp_ref[1] + 1.0
    dinv = lax.rsqrt(deg)
    dinv_ref[...] = dinv
    h = jnp.dot(x_ref[...], w_ref[...], preferred_element_type=jnp.float32)
    g_ref[...] = h * dinv[:, 0:1]


def _tc_first(x, w1, degp):
    return pl.pallas_call(
        _tc_first_body,
        grid=(GRID,),
        in_specs=[
            pl.BlockSpec((RB, D), lambda i: (i, 0)),
            pl.BlockSpec((D, H), lambda i: (0, 0)),
            pl.BlockSpec((NC, RB, L), lambda i: (0, i, 0)),
        ],
        out_specs=[
            pl.BlockSpec((RB, H), lambda i: (i, 0)),
            pl.BlockSpec((RB, L), lambda i: (i, 0)),
        ],
        out_shape=[
            jax.ShapeDtypeStruct((N, H), jnp.float32),
            jax.ShapeDtypeStruct((N, L), jnp.float32),
        ],
    )(x, w1, degp)


def _tc_combine_body(parts_ref, g_ref, dinv_ref, b_ref, w_ref, out_ref):
    dinv = dinv_ref[...][:, 0:1]
    agg = (parts_ref[0] + parts_ref[1] + g_ref[...]) * dinv + b_ref[...]
    z = jnp.maximum(agg, 0.0)
    h = jnp.dot(z, w_ref[...], preferred_element_type=jnp.float32)
    out_ref[...] = h * dinv


def _tc_combine(parts, g, dinv, b, wn):
    return pl.pallas_call(
        _tc_combine_body,
        grid=(GRID,),
        in_specs=[
            pl.BlockSpec((NC, RB, H), lambda i: (0, i, 0)),
            pl.BlockSpec((RB, H), lambda i: (i, 0)),
            pl.BlockSpec((RB, L), lambda i: (i, 0)),
            pl.BlockSpec((1, H), lambda i: (0, 0)),
            pl.BlockSpec((H, H), lambda i: (0, 0)),
        ],
        out_specs=pl.BlockSpec((RB, H), lambda i: (i, 0)),
        out_shape=jax.ShapeDtypeStruct((N, H), jnp.float32),
    )(parts, g, dinv, b, wn)


def _tc_final_body(parts_ref, g_ref, dinv_ref, b_ref,
                   dw1_ref, db1_ref, dw2_ref, db2_ref, dw3_ref, db3_ref,
                   x_ref, m_ref, z_ref, loss_ref, acc):
    i = pl.program_id(0)
    dinv = dinv_ref[...][:, 0:1]
    agg = (parts_ref[0] + parts_ref[1] + g_ref[...]) * dinv + b_ref[...]
    z = jnp.maximum(agg, 0.0)
    z_ref[...] = z
    y = jnp.maximum(
        jnp.dot(z, dw1_ref[...], preferred_element_type=jnp.float32)
        + db1_ref[...], 0.0)
    y = jnp.maximum(
        jnp.dot(y, dw2_ref[...], preferred_element_type=jnp.float32)
        + db2_ref[...], 0.0)
    xr = (jnp.dot(y, dw3_ref[...], preferred_element_type=jnp.float32)
          + db3_ref[...])
    m = m_ref[...][:, 0:1]
    part_sum = jnp.sum((xr - x_ref[...]) ** 2 * m)
    part_cnt = jnp.sum(m_ref[...][:, 0])

    @pl.when(i == 0)
    def _():
        acc[0] = 0.0
        acc[1] = 0.0

    acc[0] += part_sum
    acc[1] += part_cnt

    @pl.when(i == GRID - 1)
    def _():
        loss = acc[0] / jnp.maximum(acc[1] * jnp.float32(D), 1.0)
        loss_ref[...] = jnp.reshape(loss, (1, 1))


def _tc_final(parts, g, dinv, b3, dw1, db1, dw2, db2, dw3, db3, x, mf):
    return pl.pallas_call(
        _tc_final_body,
        grid=(GRID,),
        in_specs=[
            pl.BlockSpec((NC, RB, H), lambda i: (0, i, 0)),
            pl.BlockSpec((RB, H), lambda i: (i, 0)),
            pl.BlockSpec((RB, L), lambda i: (i, 0)),
            pl.BlockSpec((1, H), lambda i: (0, 0)),
            pl.BlockSpec((H, H), lambda i: (0, 0)),
            pl.BlockSpec((1, H), lambda i: (0, 0)),
            pl.BlockSpec((H, H), lambda i: (0, 0)),
            pl.BlockSpec((1, H), lambda i: (0, 0)),
            pl.BlockSpec((H, D), lambda i: (0, 0)),
            pl.BlockSpec((1, D), lambda i: (0, 0)),
            pl.BlockSpec((RB, D), lambda i: (i, 0)),
            pl.BlockSpec((RB, L), lambda i: (i, 0)),
        ],
        out_specs=[
            pl.BlockSpec((RB, H), lambda i: (i, 0)),
            pl.BlockSpec((1, 1), lambda i: (0, 0)),
        ],
        out_shape=[
            jax.ShapeDtypeStruct((N, H), jnp.float32),
            jax.ShapeDtypeStruct((1, 1), jnp.float32),
        ],
        scratch_shapes=[pltpu.SMEM((2,), jnp.float32)],
    )(parts, g, dinv, b3, dw1, db1, dw2, db2, dw3, db3, x, mf)


def kernel(x, edge_index, mask,
           W1, b1, W2, b2, W3, b3,
           Dw1, Db1, Dw2, Db2, Dw3, Db3):
    src = edge_index[0]
    dst = edge_index[1]

    hist = jnp.zeros((N_ACC,), jnp.float32).at[dst].add(1.0)
    degp = jnp.zeros((NC, N_ACC, L), jnp.float32)
    degp = degp.at[0].set(jnp.broadcast_to(hist[:, None], (N_ACC, L)))

    sel = jnp.array([1.0, 0.0], jnp.float32)[:, None, None]

    def agg(g):
        p = jnp.zeros((N_ACC, H), jnp.float32).at[dst].add(g[src])
        return jnp.broadcast_to(p[None], (NC, N_ACC, H)) * sel

    g, dinv = _tc_first(x, W1, degp)
    for (b, wn) in ((b1, W2), (b2, W3)):
        g = _tc_combine(agg(g), g, dinv, b.reshape(1, H), wn)
    mf = jnp.broadcast_to(mask.astype(jnp.float32)[:, None], (N, L))
    z, loss = _tc_final(agg(g), g, dinv, b3.reshape(1, H),
                        Dw1, Db1.reshape(1, H), Dw2, Db2.reshape(1, H),
                        Dw3, Db3.reshape(1, D), x, mf)
    return (loss.reshape(()), z)
